# baseline (speedup 1.0000x reference)
import functools

import jax
import jax.numpy as jnp
from jax import lax
from jax.experimental import pallas as pl
from jax.experimental.pallas import tpu as pltpu
from jax.experimental.pallas import tpu_sc as plsc

NUM_EMB = 1000000
EMB_DIM = 64
BATCH = 16384

_info = plsc.get_sparse_core_info()
_NC, _NS = _info.num_cores, _info.num_subcores
_NW = _NC * _NS
_LT = 128
_TPW = 246
_CH = 256
_CHUNKS = _TPW // 2
_NGRP = BATCH // 16
_PAIR_CAP = BATCH + 16
_SENT = 0x7FFFFFF
_NSLOT = 4
_ISEC = 4096
_ROW_B = EMB_DIM * 4


def _body(idx_hbm, tt_hbm, tail_hbm, out_hbm, idx_v, pr_v, pb_v, cpr_v,
          cpb_v, buf0_v, buf1_v, stage_v, sem_t, sem_o):
    wid = lax.axis_index("s") * _NC + lax.axis_index("c")
    lo = wid * (_TPW * _LT)
    hi = jnp.minimum(lo + _TPW * _LT, NUM_EMB)
    lanes = lax.iota(jnp.int32, 16)

    def r0_of(k):
        return pl.multiple_of(lo + k * _CH, _LT)

    def start_copy(k, buf):
        r0 = r0_of(k)
        inrange = k < _CHUNKS
        is_full = inrange & ((r0 + _CH) <= NUM_EMB)
        is_strad = inrange & (r0 < NUM_EMB) & ((r0 + _CH) > NUM_EMB)

        @pl.when(is_full)
        def _():
            pltpu.async_copy(tt_hbm.at[:, pl.ds(r0, _CH)], buf, sem_t)

        @pl.when(is_strad)
        def _():
            pltpu.async_copy(tail_hbm, buf, sem_t)

    start_copy(0, buf0_v)
    start_copy(1, buf1_v)

    M = jnp.int32(0)
    for sec in range(BATCH // _ISEC):
        pltpu.sync_copy(idx_hbm.at[pl.ds(sec * _ISEC, _ISEC)], idx_v)

        def p0(i, off, sec=sec):
            v = idx_v[pl.ds(i * 16, 16)]
            m = (v >= lo) & (v < hi)
            bv = sec * _ISEC + i * 16 + lanes
            pos = off + plsc.cumsum(jnp.where(m, 1, 0)) - 1
            plsc.store_scatter(pr_v, [pos], v, mask=m)
            plsc.store_scatter(pb_v, [pos], bv, mask=m)
            return jnp.max(pos) + 1

        M = lax.fori_loop(0, _ISEC // 16, p0, M)
    plsc.store_scatter(pr_v, [M + lanes], jnp.full((16,), _SENT, jnp.int32))
    ngr = (M + 15) // 16

    seg_v = jnp.zeros((16,), jnp.int32)
    off2 = jnp.int32(0)
    for s in range(16):
        seg_v = jnp.where(lanes == s, off2, seg_v)

        def p05(g, off, s=s):
            v = pr_v[pl.ds(g * 16, 16)]
            sid = lax.shift_right_logical(v - lo, 11)
            m = sid == s
            bv = pb_v[pl.ds(g * 16, 16)]
            pos = off + plsc.cumsum(jnp.where(m, 1, 0)) - 1
            plsc.store_scatter(cpr_v, [pos], v, mask=m)
            plsc.store_scatter(cpb_v, [pos], bv, mask=m)
            return jnp.max(pos) + 1

        off2 = lax.fori_loop(0, ngr, p05, off2)
    plsc.store_scatter(cpr_v, [off2 + lanes],
                       jnp.full((16,), _SENT, jnp.int32))

    def process(k, buf, carry, prefetch_k, pf_buf):
        bc, issued, drained, snap_v = carry
        r0 = r0_of(k)
        valid = r0 < NUM_EMB
        is_strad = valid & ((r0 + _CH) > NUM_EMB)
        rbase = jnp.where(is_strad, NUM_EMB - _CH, r0)
        rhi = jnp.minimum(r0 + _CH, NUM_EMB)

        @pl.when(valid)
        def _():
            pltpu.make_async_copy(tt_hbm.at[:, pl.ds(0, _CH)], buf,
                                  sem_t).wait()

        sc = lax.div(k, 8)
        gs = jnp.max(jnp.where(lanes == sc, seg_v, 0))
        ge = jnp.where(sc >= 15, off2,
                       jnp.max(jnp.where(lanes == sc + 1, seg_v, 0)))
        g0 = lax.div(gs, 16)
        g1 = lax.div(ge + 15, 16)

        def scan(g, cp):
            v = cpr_v[pl.ds(g * 16, 16)]
            m = (v >= r0) & (v < rhi)
            bv = cpb_v[pl.ds(g * 16, 16)]
            pos = cp + plsc.cumsum(jnp.where(m, 1, 0)) - 1
            plsc.store_scatter(pr_v, [pos], v, mask=m)
            plsc.store_scatter(pb_v, [pos], bv, mask=m)
            return jnp.max(pos) + 1

        cp = lax.fori_loop(jnp.where(valid, g0, 0),
                           jnp.where(valid, g1, 0), scan, jnp.int32(0))
        nb = (cp + 15) // 16

        def batch(t, bcarry):
            bc, issued, drained, snap_v = bcarry
            slot = lax.rem(bc, _NSLOT)
            need = jnp.max(jnp.where(lanes == slot, snap_v, 0))
            ndr = jnp.maximum(need - drained, 0)

            def drain(j, _):
                pltpu.make_async_copy(out_hbm.at[pl.ds(0, EMB_DIM)],
                                      stage_v.at[0], sem_o).wait()
                return _

            lax.fori_loop(0, ndr, drain, jnp.int32(0))
            drained = drained + ndr

            base = t * 16
            rv = pr_v[pl.ds(base, 16)]
            bv = pb_v[pl.ds(base, 16)]
            take = jnp.minimum(cp - base, 16)
            rr = jnp.where(lanes < take, rv - rbase, 0)
            srow = slot * 16 + lanes
            for c in range(EMB_DIM):
                cs = jnp.full((16,), c, jnp.int32)
                vals = plsc.load_gather(buf, [cs, rr])
                plsc.store_scatter(stage_v, [srow, cs], vals)

            def issue(j, _):
                b = jnp.max(jnp.where(lanes == j, bv, 0))
                pltpu.async_copy(
                    stage_v.at[slot * 16 + j],
                    out_hbm.at[pl.ds(pl.multiple_of(b * EMB_DIM, 8),
                                     EMB_DIM)],
                    sem_o)
                return _

            lax.fori_loop(0, take, issue, jnp.int32(0))
            issued = issued + take
            snap_v = jnp.where(lanes == slot, issued, snap_v)
            return (bc + 1, issued, drained, snap_v)

        carry = lax.fori_loop(0, nb, batch, (bc, issued, drained, snap_v))
        if prefetch_k is not None:
            start_copy(prefetch_k, pf_buf)
        return carry

    carry = (jnp.int32(0), jnp.int32(0), jnp.int32(0),
             jnp.zeros((16,), jnp.int32))
    def pairloop(g, carry):
        k0 = 2 * g
        carry = process(k0, buf0_v, carry, k0 + 2, buf0_v)
        carry = process(k0 + 1, buf1_v, carry, k0 + 3, buf1_v)
        return carry

    carry = lax.fori_loop(0, (_CHUNKS - 1) // 2, pairloop, carry)
    carry = process(_CHUNKS - 1, buf0_v, carry, None, None)

    _, issued, drained, _ = carry

    def fdrain(j, _):
        pltpu.make_async_copy(out_hbm.at[pl.ds(0, EMB_DIM)], stage_v.at[0],
                              sem_o).wait()
        return _

    lax.fori_loop(0, issued - drained, fdrain, jnp.int32(0))


@jax.jit
def kernel(indices, rules_weight):
    tt = rules_weight.T
    tail = lax.slice(rules_weight, (NUM_EMB - _CH, 0), (NUM_EMB, EMB_DIM)).T
    mesh = plsc.VectorSubcoreMesh(core_axis_name="c", subcore_axis_name="s")
    k = functools.partial(
        pl.kernel,
        mesh=mesh,
        compiler_params=pltpu.CompilerParams(needs_layout_passes=False),
        out_type=jax.ShapeDtypeStruct((BATCH * EMB_DIM,), jnp.float32),
        scratch_types=[
            pltpu.VMEM((_ISEC,), jnp.int32),
            pltpu.VMEM((_PAIR_CAP,), jnp.int32),
            pltpu.VMEM((_PAIR_CAP,), jnp.int32),
            pltpu.VMEM((_PAIR_CAP,), jnp.int32),
            pltpu.VMEM((_PAIR_CAP,), jnp.int32),
            pltpu.VMEM((EMB_DIM, _CH), jnp.float32),
            pltpu.VMEM((EMB_DIM, _CH), jnp.float32),
            pltpu.VMEM((16 * _NSLOT, EMB_DIM), jnp.float32),
            pltpu.SemaphoreType.DMA,
            pltpu.SemaphoreType.DMA,
        ],
    )(_body)
    out = k(indices, tt, tail)
    return out.reshape(BATCH, EMB_DIM)


# speedup vs baseline: 1.0062x; 1.0062x over previous
import functools

import jax
import jax.numpy as jnp
from jax import lax
from jax.experimental import pallas as pl
from jax.experimental.pallas import tpu as pltpu
from jax.experimental.pallas import tpu_sc as plsc

NUM_EMB = 1000000
EMB_DIM = 64
BATCH = 16384

_info = plsc.get_sparse_core_info()
_NC, _NS = _info.num_cores, _info.num_subcores
_NW = _NC * _NS
_LT = 128
_TPW = 246
_CH = 256
_CHUNKS = _TPW // 2
_NGRP = BATCH // 16
_PAIR_CAP = BATCH + 16
_SENT = 0x7FFFFFF
_NSLOT = 4
_ISEC = 4096
_ROW_B = EMB_DIM * 4


def _body(idx_hbm, tt_hbm, tail_hbm, out_hbm, idx_v, pr_v, pb_v, cpr_v,
          cpb_v, buf0_v, buf1_v, stage_v, tmpb_v, sem_t, sem_o):
    wid = lax.axis_index("s") * _NC + lax.axis_index("c")
    lo = wid * (_TPW * _LT)
    hi = jnp.minimum(lo + _TPW * _LT, NUM_EMB)
    lanes = lax.iota(jnp.int32, 16)

    def r0_of(k):
        return pl.multiple_of(lo + k * _CH, _LT)

    def start_copy(k, buf):
        r0 = r0_of(k)
        inrange = k < _CHUNKS
        is_full = inrange & ((r0 + _CH) <= NUM_EMB)
        is_strad = inrange & (r0 < NUM_EMB) & ((r0 + _CH) > NUM_EMB)

        @pl.when(is_full)
        def _():
            pltpu.async_copy(tt_hbm.at[:, pl.ds(r0, _CH)], buf, sem_t)

        @pl.when(is_strad)
        def _():
            pltpu.async_copy(tail_hbm, buf, sem_t)

    start_copy(0, buf0_v)
    start_copy(1, buf1_v)

    M = jnp.int32(0)
    for sec in range(BATCH // _ISEC):
        pltpu.sync_copy(idx_hbm.at[pl.ds(sec * _ISEC, _ISEC)], idx_v)

        def p0(i, off, sec=sec):
            v = idx_v[pl.ds(i * 16, 16)]
            m = (v >= lo) & (v < hi)
            bv = sec * _ISEC + i * 16 + lanes
            pos = off + plsc.cumsum(jnp.where(m, 1, 0)) - 1
            plsc.store_scatter(pr_v, [pos], v, mask=m)
            plsc.store_scatter(pb_v, [pos], bv, mask=m)
            return jnp.max(pos) + 1

        M = lax.fori_loop(0, _ISEC // 16, p0, M)
    plsc.store_scatter(pr_v, [M + lanes], jnp.full((16,), _SENT, jnp.int32))
    ngr = (M + 15) // 16

    def p05_run(s, seg_v, off2):
        seg_v = jnp.where(lanes == s, off2, seg_v)

        def p05(g, off):
            v = pr_v[pl.ds(g * 16, 16)]
            sid = lax.shift_right_logical(v - lo, 11)
            m = sid == s
            bv = pb_v[pl.ds(g * 16, 16)]
            pos = off + plsc.cumsum(jnp.where(m, 1, 0)) - 1
            plsc.store_scatter(cpr_v, [pos], v, mask=m)
            plsc.store_scatter(cpb_v, [pos], bv, mask=m)
            return jnp.max(pos) + 1

        off2 = lax.fori_loop(0, ngr, p05, off2)
        plsc.store_scatter(cpr_v, [off2 + lanes],
                           jnp.full((16,), _SENT, jnp.int32))
        return seg_v, off2

    seg_v, off2 = p05_run(jnp.int32(0), jnp.zeros((16,), jnp.int32),
                          jnp.int32(0))

    def process(k, buf, carry, prefetch_k, pf_buf):
        bc, issued, drained, snap_v, seg_v, off2 = carry
        sc = lax.div(k, 8)
        seg_v, off2 = lax.cond(
            (lax.rem(k, 8) == 0) & (sc < 15),
            lambda sv, o2: p05_run(sc + 1, sv, o2),
            lambda sv, o2: (sv, o2), seg_v, off2)
        r0 = r0_of(k)
        valid = r0 < NUM_EMB
        is_strad = valid & ((r0 + _CH) > NUM_EMB)
        rbase = jnp.where(is_strad, NUM_EMB - _CH, r0)
        rhi = jnp.minimum(r0 + _CH, NUM_EMB)

        @pl.when(valid)
        def _():
            pltpu.make_async_copy(tt_hbm.at[:, pl.ds(0, _CH)], buf,
                                  sem_t).wait()

        gs = jnp.max(jnp.where(lanes == sc, seg_v, 0))
        ge = jnp.where(sc >= 15, off2,
                       jnp.max(jnp.where(lanes == sc + 1, seg_v, 0)))
        g0 = lax.div(gs, 16)
        g1 = lax.div(ge + 15, 16)

        def batch(g, bcarry):
            bc, issued, drained, snap_v = bcarry
            rv = cpr_v[pl.ds(g * 16, 16)]
            m = (rv >= r0) & (rv < rhi)
            bv = cpb_v[pl.ds(g * 16, 16)]
            slot = lax.rem(bc, _NSLOT)
            need = jnp.max(jnp.where(lanes == slot, snap_v, 0))
            ndr = jnp.maximum(need - drained, 0)

            def drain(j, _):
                pltpu.make_async_copy(out_hbm.at[pl.ds(0, EMB_DIM)],
                                      stage_v.at[0], sem_o).wait()
                return _

            lax.fori_loop(0, ndr, drain, jnp.int32(0))
            drained = drained + ndr

            csum = plsc.cumsum(jnp.where(m, 1, 0))
            pos = jnp.where(m, csum - 1, 15)
            hits = jnp.max(csum)
            rr = jnp.where(m, rv - rbase, 0)
            srow = slot * 16 + pos
            for c in range(EMB_DIM):
                cs = jnp.full((16,), c, jnp.int32)
                vals = plsc.load_gather(buf, [cs, rr])
                plsc.store_scatter(stage_v, [srow, cs], vals, mask=m)
            plsc.store_scatter(tmpb_v, [pos], bv, mask=m)
            tb_v = tmpb_v[pl.ds(0, 16)]

            def issue(j, _):
                b = jnp.max(jnp.where(lanes == j, tb_v, 0))
                pltpu.async_copy(
                    stage_v.at[slot * 16 + j],
                    out_hbm.at[pl.ds(pl.multiple_of(b * EMB_DIM, 8),
                                     EMB_DIM)],
                    sem_o)
                return _

            lax.fori_loop(0, hits, issue, jnp.int32(0))
            issued = issued + hits
            snap_v = jnp.where(lanes == slot, issued, snap_v)
            return (bc + 1, issued, drained, snap_v)

        bc, issued, drained, snap_v = lax.fori_loop(
            jnp.where(valid, g0, 0), jnp.where(valid, g1, 0), batch,
            (bc, issued, drained, snap_v))
        if prefetch_k is not None:
            start_copy(prefetch_k, pf_buf)
        return (bc, issued, drained, snap_v, seg_v, off2)

    carry = (jnp.int32(0), jnp.int32(0), jnp.int32(0),
             jnp.zeros((16,), jnp.int32), seg_v, off2)
    def pairloop(g, carry):
        k0 = 2 * g
        carry = process(k0, buf0_v, carry, k0 + 2, buf0_v)
        carry = process(k0 + 1, buf1_v, carry, k0 + 3, buf1_v)
        return carry

    carry = lax.fori_loop(0, (_CHUNKS - 1) // 2, pairloop, carry)
    carry = process(_CHUNKS - 1, buf0_v, carry, None, None)

    _, issued, drained, _, _, _ = carry

    def fdrain(j, _):
        pltpu.make_async_copy(out_hbm.at[pl.ds(0, EMB_DIM)], stage_v.at[0],
                              sem_o).wait()
        return _

    lax.fori_loop(0, issued - drained, fdrain, jnp.int32(0))


@jax.jit
def kernel(indices, rules_weight):
    tt = rules_weight.T
    tail = lax.slice(rules_weight, (NUM_EMB - _CH, 0), (NUM_EMB, EMB_DIM)).T
    mesh = plsc.VectorSubcoreMesh(core_axis_name="c", subcore_axis_name="s")
    k = functools.partial(
        pl.kernel,
        mesh=mesh,
        compiler_params=pltpu.CompilerParams(needs_layout_passes=False),
        out_type=jax.ShapeDtypeStruct((BATCH * EMB_DIM,), jnp.float32),
        scratch_types=[
            pltpu.VMEM((_ISEC,), jnp.int32),
            pltpu.VMEM((_PAIR_CAP,), jnp.int32),
            pltpu.VMEM((_PAIR_CAP,), jnp.int32),
            pltpu.VMEM((_PAIR_CAP,), jnp.int32),
            pltpu.VMEM((_PAIR_CAP,), jnp.int32),
            pltpu.VMEM((EMB_DIM, _CH), jnp.float32),
            pltpu.VMEM((EMB_DIM, _CH), jnp.float32),
            pltpu.VMEM((16 * _NSLOT, EMB_DIM), jnp.float32),
            pltpu.VMEM((16,), jnp.int32),
            pltpu.SemaphoreType.DMA,
            pltpu.SemaphoreType.DMA,
        ],
    )(_body)
    out = k(indices, tt, tail)
    return out.reshape(BATCH, EMB_DIM)


# speedup vs baseline: 1.0729x; 1.0663x over previous
import functools

import jax
import jax.numpy as jnp
from jax import lax
from jax.experimental import pallas as pl
from jax.experimental.pallas import tpu as pltpu
from jax.experimental.pallas import tpu_sc as plsc

NUM_EMB = 1000000
EMB_DIM = 64
BATCH = 16384

_info = plsc.get_sparse_core_info()
_NC, _NS = _info.num_cores, _info.num_subcores
_NW = _NC * _NS
_LT = 128
_TPW = 246
_CH = 256
_CHUNKS = _TPW // 2
_NGRP = BATCH // 16
_PAIR_CAP = BATCH + 16
_SENT = 0x7FFFFFF
_NSLOT = 4
_ISEC = 2048
_ROW_B = EMB_DIM * 4


def _body(idx_hbm, tt_hbm, tail_hbm, out_hbm, idx_v, pr_v, pb_v, cpr_v,
          cpb_v, buf0_v, buf1_v, buf2_v, stage_v, tmpb_v, sem_t, sem_o):
    wid = lax.axis_index("s") * _NC + lax.axis_index("c")
    lo = wid * (_TPW * _LT)
    hi = jnp.minimum(lo + _TPW * _LT, NUM_EMB)
    lanes = lax.iota(jnp.int32, 16)

    def r0_of(k):
        return pl.multiple_of(lo + k * _CH, _LT)

    def start_copy(k, buf):
        r0 = r0_of(k)
        inrange = k < _CHUNKS
        is_full = inrange & ((r0 + _CH) <= NUM_EMB)
        is_strad = inrange & (r0 < NUM_EMB) & ((r0 + _CH) > NUM_EMB)

        @pl.when(is_full)
        def _():
            pltpu.async_copy(tt_hbm.at[:, pl.ds(r0, _CH)], buf, sem_t)

        @pl.when(is_strad)
        def _():
            pltpu.async_copy(tail_hbm, buf, sem_t)

    start_copy(0, buf0_v)
    start_copy(1, buf1_v)
    start_copy(2, buf2_v)

    M = jnp.int32(0)
    for sec in range(BATCH // _ISEC):
        pltpu.sync_copy(idx_hbm.at[pl.ds(sec * _ISEC, _ISEC)], idx_v)

        def p0(i, off, sec=sec):
            v = idx_v[pl.ds(i * 16, 16)]
            m = (v >= lo) & (v < hi)
            bv = sec * _ISEC + i * 16 + lanes
            pos = off + plsc.cumsum(jnp.where(m, 1, 0)) - 1
            plsc.store_scatter(pr_v, [pos], v, mask=m)
            plsc.store_scatter(pb_v, [pos], bv, mask=m)
            return jnp.max(pos) + 1

        M = lax.fori_loop(0, _ISEC // 16, p0, M)
    plsc.store_scatter(pr_v, [M + lanes], jnp.full((16,), _SENT, jnp.int32))
    ngr = (M + 15) // 16

    def p05_run(s, seg_v, off2):
        seg_v = jnp.where(lanes == s, off2, seg_v)

        def p05(g, off):
            v = pr_v[pl.ds(g * 16, 16)]
            sid = lax.shift_right_logical(v - lo, 11)
            m = sid == s
            bv = pb_v[pl.ds(g * 16, 16)]
            pos = off + plsc.cumsum(jnp.where(m, 1, 0)) - 1
            plsc.store_scatter(cpr_v, [pos], v, mask=m)
            plsc.store_scatter(cpb_v, [pos], bv, mask=m)
            return jnp.max(pos) + 1

        off2 = lax.fori_loop(0, ngr, p05, off2)
        plsc.store_scatter(cpr_v, [off2 + lanes],
                           jnp.full((16,), _SENT, jnp.int32))
        return seg_v, off2

    seg_v, off2 = p05_run(jnp.int32(0), jnp.zeros((16,), jnp.int32),
                          jnp.int32(0))

    def process(k, buf, carry, prefetch_k, pf_buf):
        bc, issued, drained, snap_v, seg_v, off2 = carry
        sc = lax.div(k, 8)
        seg_v, off2 = lax.cond(
            (lax.rem(k, 8) == 0) & (sc < 15),
            lambda sv, o2: p05_run(sc + 1, sv, o2),
            lambda sv, o2: (sv, o2), seg_v, off2)
        r0 = r0_of(k)
        valid = r0 < NUM_EMB
        is_strad = valid & ((r0 + _CH) > NUM_EMB)
        rbase = jnp.where(is_strad, NUM_EMB - _CH, r0)
        rhi = jnp.minimum(r0 + _CH, NUM_EMB)

        @pl.when(valid)
        def _():
            pltpu.make_async_copy(tt_hbm.at[:, pl.ds(0, _CH)], buf,
                                  sem_t).wait()

        gs = jnp.max(jnp.where(lanes == sc, seg_v, 0))
        ge = jnp.where(sc >= 15, off2,
                       jnp.max(jnp.where(lanes == sc + 1, seg_v, 0)))
        g0 = lax.div(gs, 16)
        g1 = lax.div(ge + 15, 16)

        def batch(g, bcarry):
            bc, issued, drained, snap_v = bcarry
            rv = cpr_v[pl.ds(g * 16, 16)]
            m = (rv >= r0) & (rv < rhi)
            bv = cpb_v[pl.ds(g * 16, 16)]
            slot = lax.rem(bc, _NSLOT)
            need = jnp.max(jnp.where(lanes == slot, snap_v, 0))
            ndr = jnp.maximum(need - drained, 0)

            def drain(j, _):
                pltpu.make_async_copy(out_hbm.at[pl.ds(0, EMB_DIM)],
                                      stage_v.at[0], sem_o).wait()
                return _

            lax.fori_loop(0, ndr, drain, jnp.int32(0))
            drained = drained + ndr

            csum = plsc.cumsum(jnp.where(m, 1, 0))
            pos = jnp.where(m, csum - 1, 15)
            hits = jnp.max(csum)
            rr = jnp.where(m, rv - rbase, 0)
            srow = slot * 16 + pos
            for c in range(EMB_DIM):
                cs = jnp.full((16,), c, jnp.int32)
                vals = plsc.load_gather(buf, [cs, rr])
                plsc.store_scatter(stage_v, [srow, cs], vals, mask=m)
            plsc.store_scatter(tmpb_v, [pos], bv, mask=m)
            tb_v = tmpb_v[pl.ds(0, 16)]

            def issue(j, _):
                b = jnp.max(jnp.where(lanes == j, tb_v, 0))
                pltpu.async_copy(
                    stage_v.at[slot * 16 + j],
                    out_hbm.at[pl.ds(pl.multiple_of(b * EMB_DIM, 8),
                                     EMB_DIM)],
                    sem_o)
                return _

            lax.fori_loop(0, hits, issue, jnp.int32(0))
            issued = issued + hits
            snap_v = jnp.where(lanes == slot, issued, snap_v)
            return (bc + 1, issued, drained, snap_v)

        bc, issued, drained, snap_v = lax.fori_loop(
            jnp.where(valid, g0, 0), jnp.where(valid, g1, 0), batch,
            (bc, issued, drained, snap_v))
        if prefetch_k is not None:
            start_copy(prefetch_k, pf_buf)
        return (bc, issued, drained, snap_v, seg_v, off2)

    carry = (jnp.int32(0), jnp.int32(0), jnp.int32(0),
             jnp.zeros((16,), jnp.int32), seg_v, off2)
    def pairloop(g, carry):
        k0 = 3 * g
        carry = process(k0, buf0_v, carry, k0 + 3, buf0_v)
        carry = process(k0 + 1, buf1_v, carry, k0 + 4, buf1_v)
        carry = process(k0 + 2, buf2_v, carry, k0 + 5, buf2_v)
        return carry

    carry = lax.fori_loop(0, _CHUNKS // 3, pairloop, carry)

    _, issued, drained, _, _, _ = carry

    def fdrain(j, _):
        pltpu.make_async_copy(out_hbm.at[pl.ds(0, EMB_DIM)], stage_v.at[0],
                              sem_o).wait()
        return _

    lax.fori_loop(0, issued - drained, fdrain, jnp.int32(0))


@jax.jit
def kernel(indices, rules_weight):
    tt = rules_weight.T
    tail = lax.slice(rules_weight, (NUM_EMB - _CH, 0), (NUM_EMB, EMB_DIM)).T
    mesh = plsc.VectorSubcoreMesh(core_axis_name="c", subcore_axis_name="s")
    k = functools.partial(
        pl.kernel,
        mesh=mesh,
        compiler_params=pltpu.CompilerParams(needs_layout_passes=False),
        out_type=jax.ShapeDtypeStruct((BATCH * EMB_DIM,), jnp.float32),
        scratch_types=[
            pltpu.VMEM((_ISEC,), jnp.int32),
            pltpu.VMEM((_PAIR_CAP,), jnp.int32),
            pltpu.VMEM((_PAIR_CAP,), jnp.int32),
            pltpu.VMEM((_PAIR_CAP,), jnp.int32),
            pltpu.VMEM((_PAIR_CAP,), jnp.int32),
            pltpu.VMEM((EMB_DIM, _CH), jnp.float32),
            pltpu.VMEM((EMB_DIM, _CH), jnp.float32),
            pltpu.VMEM((EMB_DIM, _CH), jnp.float32),
            pltpu.VMEM((16 * _NSLOT, EMB_DIM), jnp.float32),
            pltpu.VMEM((16,), jnp.int32),
            pltpu.SemaphoreType.DMA,
            pltpu.SemaphoreType.DMA,
        ],
    )(_body)
    out = k(indices, tt, tail)
    return out.reshape(BATCH, EMB_DIM)


# speedup vs baseline: 1.4027x; 1.3074x over previous
import functools

import jax
import jax.numpy as jnp
from jax import lax
from jax.experimental import pallas as pl
from jax.experimental.pallas import tpu as pltpu
from jax.experimental.pallas import tpu_sc as plsc

NUM_EMB = 1000000
EMB_DIM = 64
BATCH = 16384

_info = plsc.get_sparse_core_info()
_NC, _NS = _info.num_cores, _info.num_subcores
_NW = _NC * _NS
_LT = 128
_TPW = 246
_CH = 256
_CHUNKS = _TPW // 2
_NGRP = BATCH // 16
_PAIR_CAP = BATCH + 16
_SENT = 0x7FFFFFF
_NSLOT = 4
_ISEC = 2048
_ROW_B = EMB_DIM * 4


def _body(idx_hbm, tt_hbm, tail_hbm, out_hbm, idx_v, pr_v, pb_v, cpr_v,
          cpb_v, buf0_v, buf1_v, buf2_v, stage_v, tmpb_v, tmpr_v, sem_t, sem_o):
    wid = lax.axis_index("s") * _NC + lax.axis_index("c")
    lo = wid * (_TPW * _LT)
    hi = jnp.minimum(lo + _TPW * _LT, NUM_EMB)
    lanes = lax.iota(jnp.int32, 16)

    def r0_of(k):
        return pl.multiple_of(lo + k * _CH, _LT)

    def start_copy(k, buf):
        r0 = r0_of(k)
        inrange = k < _CHUNKS
        is_full = inrange & ((r0 + _CH) <= NUM_EMB)
        is_strad = inrange & (r0 < NUM_EMB) & ((r0 + _CH) > NUM_EMB)

        @pl.when(is_full)
        def _():
            pltpu.async_copy(tt_hbm.at[:, pl.ds(r0, _CH)], buf, sem_t)

        @pl.when(is_strad)
        def _():
            pltpu.async_copy(tail_hbm, buf, sem_t)

    start_copy(0, buf0_v)
    start_copy(1, buf1_v)
    start_copy(2, buf2_v)

    M = jnp.int32(0)
    for sec in range(BATCH // _ISEC):
        pltpu.sync_copy(idx_hbm.at[pl.ds(sec * _ISEC, _ISEC)], idx_v)

        def p0(i, off, sec=sec):
            v = idx_v[pl.ds(i * 16, 16)]
            m = (v >= lo) & (v < hi)
            bv = sec * _ISEC + i * 16 + lanes
            pos = off + plsc.cumsum(jnp.where(m, 1, 0)) - 1
            plsc.store_scatter(pr_v, [pos], v, mask=m)
            plsc.store_scatter(pb_v, [pos], bv, mask=m)
            return jnp.max(pos) + 1

        M = lax.fori_loop(0, _ISEC // 16, p0, M)
    plsc.store_scatter(pr_v, [M + lanes], jnp.full((16,), _SENT, jnp.int32))
    ngr = (M + 15) // 16

    def p05_run(s, seg_v, off2):
        seg_v = jnp.where(lanes == s, off2, seg_v)

        def p05(g, off):
            v = pr_v[pl.ds(g * 16, 16)]
            sid = lax.shift_right_logical(v - lo, 11)
            m = sid == s
            bv = pb_v[pl.ds(g * 16, 16)]
            pos = off + plsc.cumsum(jnp.where(m, 1, 0)) - 1
            plsc.store_scatter(cpr_v, [pos], v, mask=m)
            plsc.store_scatter(cpb_v, [pos], bv, mask=m)
            return jnp.max(pos) + 1

        off2 = lax.fori_loop(0, ngr, p05, off2)
        plsc.store_scatter(cpr_v, [off2 + lanes],
                           jnp.full((16,), _SENT, jnp.int32))
        return seg_v, off2

    seg_v, off2 = p05_run(jnp.int32(0), jnp.zeros((16,), jnp.int32),
                          jnp.int32(0))

    def process(k, buf, carry, prefetch_k, pf_buf):
        bc, issued, drained, snap_v, seg_v, off2 = carry
        sc = lax.div(k, 8)
        seg_v, off2 = lax.cond(
            (lax.rem(k, 8) == 0) & (sc < 15),
            lambda sv, o2: p05_run(sc + 1, sv, o2),
            lambda sv, o2: (sv, o2), seg_v, off2)
        r0 = r0_of(k)
        valid = r0 < NUM_EMB
        is_strad = valid & ((r0 + _CH) > NUM_EMB)
        rbase = jnp.where(is_strad, NUM_EMB - _CH, r0)
        rhi = jnp.minimum(r0 + _CH, NUM_EMB)

        @pl.when(valid)
        def _():
            pltpu.make_async_copy(tt_hbm.at[:, pl.ds(0, _CH)], buf,
                                  sem_t).wait()

        gs = jnp.max(jnp.where(lanes == sc, seg_v, 0))
        ge = jnp.where(sc >= 15, off2,
                       jnp.max(jnp.where(lanes == sc + 1, seg_v, 0)))
        g0 = lax.div(gs, 16)
        g1 = lax.div(ge + 15, 16)

        def batch(g, bcarry):
            bc, issued, drained, snap_v = bcarry
            rv = cpr_v[pl.ds(g * 16, 16)]
            m = (rv >= r0) & (rv < rhi)
            bv = cpb_v[pl.ds(g * 16, 16)]
            slot = lax.rem(bc, _NSLOT)
            need = jnp.max(jnp.where(lanes == slot, snap_v, 0))
            ndr = jnp.maximum(need - drained, 0)

            def drain(j, _):
                pltpu.make_async_copy(out_hbm.at[pl.ds(0, EMB_DIM)],
                                      stage_v.at[0], sem_o).wait()
                return _

            lax.fori_loop(0, ndr, drain, jnp.int32(0))
            drained = drained + ndr

            csum = plsc.cumsum(jnp.where(m, 1, 0))
            pos = jnp.where(m, csum - 1, 15)
            hits = jnp.max(csum)
            plsc.store_scatter(tmpr_v, [pos], rv - rbase, mask=m)
            plsc.store_scatter(tmpb_v, [pos], bv, mask=m)
            tr_v = tmpr_v[pl.ds(0, 16)]
            tb_v = tmpb_v[pl.ds(0, 16)]

            def issue(j, _):
                rrj = jnp.max(jnp.where(lanes == j, tr_v, 0))
                b = jnp.max(jnp.where(lanes == j, tb_v, 0))
                row = jnp.full((16,), slot * 16, jnp.int32) + j
                rrs = jnp.full((16,), 0, jnp.int32) + rrj
                for q in range(EMB_DIM // 16):
                    cs = q * 16 + lanes
                    vals = plsc.load_gather(buf, [cs, rrs])
                    plsc.store_scatter(stage_v, [row, cs], vals)
                pltpu.async_copy(
                    stage_v.at[slot * 16 + j],
                    out_hbm.at[pl.ds(pl.multiple_of(b * EMB_DIM, 8),
                                     EMB_DIM)],
                    sem_o)
                return _

            lax.fori_loop(0, hits, issue, jnp.int32(0))
            issued = issued + hits
            snap_v = jnp.where(lanes == slot, issued, snap_v)
            return (bc + 1, issued, drained, snap_v)

        bc, issued, drained, snap_v = lax.fori_loop(
            jnp.where(valid, g0, 0), jnp.where(valid, g1, 0), batch,
            (bc, issued, drained, snap_v))
        if prefetch_k is not None:
            start_copy(prefetch_k, pf_buf)
        return (bc, issued, drained, snap_v, seg_v, off2)

    carry = (jnp.int32(0), jnp.int32(0), jnp.int32(0),
             jnp.zeros((16,), jnp.int32), seg_v, off2)
    def pairloop(g, carry):
        k0 = 3 * g
        carry = process(k0, buf0_v, carry, k0 + 3, buf0_v)
        carry = process(k0 + 1, buf1_v, carry, k0 + 4, buf1_v)
        carry = process(k0 + 2, buf2_v, carry, k0 + 5, buf2_v)
        return carry

    carry = lax.fori_loop(0, _CHUNKS // 3, pairloop, carry)

    _, issued, drained, _, _, _ = carry

    def fdrain(j, _):
        pltpu.make_async_copy(out_hbm.at[pl.ds(0, EMB_DIM)], stage_v.at[0],
                              sem_o).wait()
        return _

    lax.fori_loop(0, issued - drained, fdrain, jnp.int32(0))


@jax.jit
def kernel(indices, rules_weight):
    tt = rules_weight.T
    tail = lax.slice(rules_weight, (NUM_EMB - _CH, 0), (NUM_EMB, EMB_DIM)).T
    mesh = plsc.VectorSubcoreMesh(core_axis_name="c", subcore_axis_name="s")
    k = functools.partial(
        pl.kernel,
        mesh=mesh,
        compiler_params=pltpu.CompilerParams(needs_layout_passes=False),
        out_type=jax.ShapeDtypeStruct((BATCH * EMB_DIM,), jnp.float32),
        scratch_types=[
            pltpu.VMEM((_ISEC,), jnp.int32),
            pltpu.VMEM((_PAIR_CAP,), jnp.int32),
            pltpu.VMEM((_PAIR_CAP,), jnp.int32),
            pltpu.VMEM((_PAIR_CAP,), jnp.int32),
            pltpu.VMEM((_PAIR_CAP,), jnp.int32),
            pltpu.VMEM((EMB_DIM, _CH), jnp.float32),
            pltpu.VMEM((EMB_DIM, _CH), jnp.float32),
            pltpu.VMEM((EMB_DIM, _CH), jnp.float32),
            pltpu.VMEM((16 * _NSLOT, EMB_DIM), jnp.float32),
            pltpu.VMEM((16,), jnp.int32),
            pltpu.VMEM((16,), jnp.int32),
            pltpu.SemaphoreType.DMA,
            pltpu.SemaphoreType.DMA,
        ],
    )(_body)
    out = k(indices, tt, tail)
    return out.reshape(BATCH, EMB_DIM)
